# baseline (speedup 1.0000x reference)
import functools as _functools

import jax
import jax.numpy as jnp
import numpy as np
from jax import lax
from jax.experimental import pallas as pl
from jax.experimental.pallas import tpu as pltpu
from jax.experimental.pallas import tpu_sc as plsc

D = 59
C_OUT = 64
HF, WF = 16, 44
NPIX = HF * WF
NIMG = 12
NPIX_ALL = NIMG * NPIX
NVOX = 16384
NJUNK = 1024
NV = NVOX + NJUNK
DPAD = 64

DXV = np.array([0.8, 0.8, 20.0], dtype=np.float32)
BXV = np.array([-50.8, -50.8, 0.0], dtype=np.float32)

NCORE = 2
NTILE = 16
NPIX_PAD = 8704
PIX_PER_CORE = NPIX_PAD // NCORE
PIX_PER_TILE = PIX_PER_CORE // NTILE
PCHUNK = 8
NCHUNK = PIX_PER_TILE // PCHUNK
RKV_ROWS = PCHUNK * DPAD // 128
ROWS_PER_TILE = NV // NTILE


def _prep_body(img_ref, w_ref, b_ref, g_ref, wt_ref, rk_ref):
    img = img_ref[0]
    w = w_ref[...]
    xt = lax.dot_general(img, w, (((0,), (1,)), ((), ())),
                         preferred_element_type=jnp.float32)
    xt = xt + b_ref[...]
    cc = lax.broadcasted_iota(jnp.int32, (NPIX, 128), 1)
    maskD = cc < D
    m = jnp.max(jnp.where(maskD, xt, jnp.float32(-1e30)), axis=1, keepdims=True)
    e = jnp.where(maskD, jnp.exp(xt - m), jnp.float32(0.0))
    depth = e / jnp.sum(e, axis=1, keepdims=True)
    wt_ref[...] = jnp.concatenate([depth[:, :C_OUT], xt[:, D:D + C_OUT]], axis=1)

    gx = g_ref[0, 0]
    gy = g_ref[0, 1]
    gz = g_ref[0, 2]
    ix = ((gx - jnp.float32(BXV[0])) / jnp.float32(DXV[0])).astype(jnp.int32)
    iy = ((gy - jnp.float32(BXV[1])) / jnp.float32(DXV[1])).astype(jnp.int32)
    iz = ((gz - jnp.float32(BXV[2])) / jnp.float32(DXV[2])).astype(jnp.int32)
    pp = lax.broadcasted_iota(jnp.int32, (NPIX, DPAD), 0)
    jj = lax.broadcasted_iota(jnp.int32, (NPIX, DPAD), 1)
    valid = ((ix >= 0) & (ix < 128) & (iy >= 0) & (iy < 128)
             & (iz >= 0) & (iz < 1) & (jj < D))
    rank = ix + iy * 128 + iz * (128 * 128)
    junk = NVOX + (pp % 16) * 64 + jj
    rk_ref[...] = jnp.where(valid, rank, junk)


_prep = pl.pallas_call(
    _prep_body,
    grid=(NIMG,),
    in_specs=[
        pl.BlockSpec((1, 512, NPIX), lambda i: (i, 0, 0)),
        pl.BlockSpec((128, 512), lambda i: (0, 0)),
        pl.BlockSpec((1, 128), lambda i: (0, 0)),
        pl.BlockSpec((1, 3, NPIX, DPAD), lambda i: (i, 0, 0, 0)),
    ],
    out_specs=[
        pl.BlockSpec((NPIX, 128), lambda i: (i, 0)),
        pl.BlockSpec((NPIX, DPAD), lambda i: (i, 0)),
    ],
    out_shape=[
        jax.ShapeDtypeStruct((NPIX_ALL, 128), jnp.float32),
        jax.ShapeDtypeStruct((NPIX_ALL, DPAD), jnp.int32),
    ],
)


def _geometry(rots, trans, intrins, post_rots, post_trans):
    xs = jnp.linspace(0.0, WF - 1.0, WF, dtype=jnp.float32).reshape(1, WF, 1) * jnp.ones(
        (HF, 1, DPAD), jnp.float32)
    ys = jnp.linspace(0.0, HF - 1.0, HF, dtype=jnp.float32).reshape(HF, 1, 1) * jnp.ones(
        (1, WF, DPAD), jnp.float32)
    ds_ = jnp.arange(1.0, DPAD + 1.0, 1.0, dtype=jnp.float32).reshape(1, 1, DPAD) * jnp.ones(
        (HF, WF, 1), jnp.float32)
    frustum = jnp.stack((xs, ys, ds_), -1)
    points = frustum[None, None] - post_trans[:, :, None, None, None, :]
    inv_pr = jnp.linalg.inv(post_rots)
    points = jnp.einsum('bnij,bnhwdj->bnihwd', inv_pr, points)
    uv = points[:, :, :2]
    dd = points[:, :, 2:3]
    points = jnp.concatenate((uv * dd, dd), axis=2)
    combine = jnp.matmul(rots, jnp.linalg.inv(intrins))
    points = jnp.einsum('bnij,bnjhwd->bnihwd', combine, points)
    points = points + trans[:, :, :, None, None, None]
    return points.reshape(NIMG, 3, NPIX, DPAD)


def _sc_body(wt_hbm, rk_hbm, out_hbm, bev, wtbuf, rkbuf, s0, s1, s2, s3,
             outbuf, sem):
    c = lax.axis_index("c")
    s = lax.axis_index("s")
    bufs = [s0, s1, s2, s3]

    def zrow(r, _):
        z = jnp.zeros((16,), jnp.float32)
        for j in range(4):
            s0[r, pl.ds(16 * j, 16)] = z
        return 0
    lax.fori_loop(0, 128, zrow, 0)

    def zcpy(k, _):
        pltpu.sync_copy(s0.at[pl.ds(0, DPAD)],
                        bev.at[pl.ds(s * ROWS_PER_TILE + k * DPAD, DPAD)])
        return 0
    lax.fori_loop(0, ROWS_PER_TILE // DPAD, zcpy, 0)
    plsc.subcore_barrier()

    pixbase0 = c * PIX_PER_CORE + s * PIX_PER_TILE

    def chunk(ch, _):
        pixb = pixbase0 + ch * PCHUNK

        @pl.when(pixb < NPIX_ALL)
        def _do_chunk():
            pltpu.sync_copy(wt_hbm.at[pl.ds(pixb, PCHUNK)], wtbuf)
            pltpu.sync_copy(rk_hbm.at[pl.ds(pixb // 2 * (DPAD // 64), RKV_ROWS)],
                            rkbuf)
            cps = []
            for q in range(4):
                buf = bufs[q]
                for half in range(2):
                    i = q * 2 + half
                    t0 = wtbuf[i, pl.ds(64, 16)]
                    t1 = wtbuf[i, pl.ds(80, 16)]
                    t2 = wtbuf[i, pl.ds(96, 16)]
                    t3 = wtbuf[i, pl.ds(112, 16)]
                    wv = [wtbuf[i, pl.ds(16 * g, 16)] for g in range(4)]
                    for dd in range(DPAD):
                        wgt = wv[dd // 16][dd % 16]
                        row = half * DPAD + dd
                        buf[row, pl.ds(0, 16)] = wgt * t0
                        buf[row, pl.ds(16, 16)] = wgt * t1
                        buf[row, pl.ds(32, 16)] = wgt * t2
                        buf[row, pl.ds(48, 16)] = wgt * t3
                cps.append(pltpu.async_copy(buf, bev.at[rkbuf.at[q]], sem,
                                            add=True))
            for cp in cps:
                cp.wait()
        return 0
    lax.fori_loop(0, NCHUNK, chunk, 0)

    plsc.subcore_barrier()

    obase = s * (ROWS_PER_TILE * C_OUT // 128)

    def rd(k, _):
        pltpu.sync_copy(bev.at[pl.ds(s * ROWS_PER_TILE + k * DPAD, DPAD)],
                        s0.at[pl.ds(0, DPAD)])

        def rt(rr, _):
            for h in range(2):
                for j in range(4):
                    outbuf[rr, pl.ds(h * 64 + 16 * j, 16)] = (
                        s0[rr * 2 + h, pl.ds(16 * j, 16)])
            return 0
        lax.fori_loop(0, 32, rt, 0)
        pltpu.sync_copy(outbuf, out_hbm.at[c, pl.ds(obase + k * 32, 32)])
        return 0
    lax.fori_loop(0, ROWS_PER_TILE // DPAD, rd, 0)


@_functools.cache
def _get_scatter():
    return pl.kernel(
        _sc_body,
        out_type=jax.ShapeDtypeStruct((NCORE, NV * C_OUT // 128, 128), jnp.float32),
        mesh=plsc.VectorSubcoreMesh(core_axis_name="c", subcore_axis_name="s",
                                    num_cores=NCORE, num_subcores=NTILE),
        compiler_params=pltpu.CompilerParams(use_tc_tiling_on_sc=False),
        scratch_types=[
            pltpu.VMEM_SHARED((NV, C_OUT), jnp.float32),
            pltpu.VMEM((PCHUNK, 128), jnp.float32),
            pltpu.VMEM((RKV_ROWS, 128), jnp.int32),
            pltpu.VMEM((128, C_OUT), jnp.float32),
            pltpu.VMEM((128, C_OUT), jnp.float32),
            pltpu.VMEM((128, C_OUT), jnp.float32),
            pltpu.VMEM((128, C_OUT), jnp.float32),
            pltpu.VMEM((32, 128), jnp.float32),
            pltpu.SemaphoreType.DMA,
        ],
    )


def _comb_body(parts_ref, eye_ref, out_ref):
    p = parts_ref[...]
    ssum = p[0] + p[1]
    out_ref[...] = lax.dot_general(eye_ref[...], ssum, (((1,), (1,)), ((), ())),
                                   preferred_element_type=jnp.float32)


_comb = pl.pallas_call(
    _comb_body,
    grid=(8,),
    in_specs=[
        pl.BlockSpec((NCORE, 2048, C_OUT), lambda k: (0, k, 0)),
        pl.BlockSpec((C_OUT, C_OUT), lambda k: (0, 0)),
    ],
    out_specs=pl.BlockSpec((C_OUT, 2048), lambda k: (0, k)),
    out_shape=jax.ShapeDtypeStruct((C_OUT, NVOX), jnp.float32),
)


def kernel(img_feats, rots, trans, intrins, post_rots, post_trans, W, b):
    B, N = img_feats.shape[:2]
    img = img_feats.reshape(B * N, 512, NPIX)
    Wp = jnp.zeros((128, 512), jnp.float32).at[:D + C_OUT].set(W)
    bp = jnp.zeros((1, 128), jnp.float32).at[0, :D + C_OUT].set(b)
    geom = _geometry(rots, trans, intrins, post_rots, post_trans)
    wt, rk = _prep(img, Wp, bp, geom)
    rkv = rk.reshape(NPIX_ALL * DPAD // 128, 128)
    parts = _get_scatter()(wt, rkv)
    parts = parts.reshape(NCORE, NV, C_OUT)
    eye = jnp.eye(C_OUT, dtype=jnp.float32)
    flat = _comb(parts, eye)
    return flat.reshape(1, C_OUT, 128, 128)


# speedup vs baseline: 1.1553x; 1.1553x over previous
import functools as _functools

import jax
import jax.numpy as jnp
import numpy as np
from jax import lax
from jax.experimental import pallas as pl
from jax.experimental.pallas import tpu as pltpu
from jax.experimental.pallas import tpu_sc as plsc

D = 59
C_OUT = 64
HF, WF = 16, 44
NPIX = HF * WF
NIMG = 12
NPIX_ALL = NIMG * NPIX
NVOX = 16384
NJUNK = 1024
NV = NVOX + NJUNK
DPAD = 64

DXV = np.array([0.8, 0.8, 20.0], dtype=np.float32)
BXV = np.array([-50.8, -50.8, 0.0], dtype=np.float32)

NCORE = 2
NTILE = 16
NPIX_PAD = 8704
PIX_PER_CORE = NPIX_PAD // NCORE
PIX_PER_TILE = PIX_PER_CORE // NTILE
PCHUNK = 16
NCHUNK = PIX_PER_TILE // PCHUNK
RKV_ROWS = PCHUNK * DPAD // 128
ROWS_PER_TILE = NV // NTILE


def _prep_body(img_ref, w_ref, b_ref, g_ref, wt_ref, rk_ref):
    img = img_ref[0]
    w = w_ref[...]
    xt = lax.dot_general(img, w, (((0,), (1,)), ((), ())),
                         preferred_element_type=jnp.float32)
    xt = xt + b_ref[...]
    cc = lax.broadcasted_iota(jnp.int32, (NPIX, 128), 1)
    maskD = cc < D
    m = jnp.max(jnp.where(maskD, xt, jnp.float32(-1e30)), axis=1, keepdims=True)
    e = jnp.where(maskD, jnp.exp(xt - m), jnp.float32(0.0))
    depth = e / jnp.sum(e, axis=1, keepdims=True)
    wt_ref[...] = jnp.concatenate([depth[:, :C_OUT], xt[:, D:D + C_OUT]], axis=1)

    gx = g_ref[0, 0]
    gy = g_ref[0, 1]
    gz = g_ref[0, 2]
    ix = ((gx - jnp.float32(BXV[0])) / jnp.float32(DXV[0])).astype(jnp.int32)
    iy = ((gy - jnp.float32(BXV[1])) / jnp.float32(DXV[1])).astype(jnp.int32)
    iz = ((gz - jnp.float32(BXV[2])) / jnp.float32(DXV[2])).astype(jnp.int32)
    pp = lax.broadcasted_iota(jnp.int32, (NPIX, DPAD), 0)
    jj = lax.broadcasted_iota(jnp.int32, (NPIX, DPAD), 1)
    valid = ((ix >= 0) & (ix < 128) & (iy >= 0) & (iy < 128)
             & (iz >= 0) & (iz < 1) & (jj < D))
    rank = ix + iy * 128 + iz * (128 * 128)
    junk = NVOX + (pp % 16) * 64 + jj
    rk_ref[...] = jnp.where(valid, rank, junk)


_prep = pl.pallas_call(
    _prep_body,
    grid=(NIMG,),
    in_specs=[
        pl.BlockSpec((1, 512, NPIX), lambda i: (i, 0, 0)),
        pl.BlockSpec((128, 512), lambda i: (0, 0)),
        pl.BlockSpec((1, 128), lambda i: (0, 0)),
        pl.BlockSpec((1, 3, NPIX, DPAD), lambda i: (i, 0, 0, 0)),
    ],
    out_specs=[
        pl.BlockSpec((NPIX, 128), lambda i: (i, 0)),
        pl.BlockSpec((NPIX, DPAD), lambda i: (i, 0)),
    ],
    out_shape=[
        jax.ShapeDtypeStruct((NPIX_ALL, 128), jnp.float32),
        jax.ShapeDtypeStruct((NPIX_ALL, DPAD), jnp.int32),
    ],
)


def _geometry(rots, trans, intrins, post_rots, post_trans):
    xs = jnp.linspace(0.0, WF - 1.0, WF, dtype=jnp.float32).reshape(1, WF, 1) * jnp.ones(
        (HF, 1, DPAD), jnp.float32)
    ys = jnp.linspace(0.0, HF - 1.0, HF, dtype=jnp.float32).reshape(HF, 1, 1) * jnp.ones(
        (1, WF, DPAD), jnp.float32)
    ds_ = jnp.arange(1.0, DPAD + 1.0, 1.0, dtype=jnp.float32).reshape(1, 1, DPAD) * jnp.ones(
        (HF, WF, 1), jnp.float32)
    frustum = jnp.stack((xs, ys, ds_), -1)
    points = frustum[None, None] - post_trans[:, :, None, None, None, :]
    inv_pr = jnp.linalg.inv(post_rots)
    points = jnp.einsum('bnij,bnhwdj->bnihwd', inv_pr, points)
    uv = points[:, :, :2]
    dd = points[:, :, 2:3]
    points = jnp.concatenate((uv * dd, dd), axis=2)
    combine = jnp.matmul(rots, jnp.linalg.inv(intrins))
    points = jnp.einsum('bnij,bnjhwd->bnihwd', combine, points)
    points = points + trans[:, :, :, None, None, None]
    return points.reshape(NIMG, 3, NPIX, DPAD)


def _sc_body(wt_hbm, rk_hbm, out_hbm, bev, wtbuf, rkbuf, scaled, outbuf):
    c = lax.axis_index("c")
    s = lax.axis_index("s")

    def zrow(r, _):
        z = jnp.zeros((16,), jnp.float32)
        for j in range(4):
            scaled[r, pl.ds(16 * j, 16)] = z
        return 0
    lax.fori_loop(0, 128, zrow, 0)

    def zcpy(k, _):
        pltpu.sync_copy(scaled.at[pl.ds(0, DPAD)],
                        bev.at[pl.ds(s * ROWS_PER_TILE + k * DPAD, DPAD)])
        return 0
    lax.fori_loop(0, ROWS_PER_TILE // DPAD, zcpy, 0)
    plsc.subcore_barrier()

    pixbase0 = c * PIX_PER_CORE + s * PIX_PER_TILE

    def chunk(ch, _):
        pixb = pixbase0 + ch * PCHUNK

        @pl.when(pixb < NPIX_ALL)
        def _do_chunk():
            _chunk_body(pixb)
        return 0

    def _chunk_body(pixb):
        pltpu.sync_copy(wt_hbm.at[pl.ds(pixb, PCHUNK)], wtbuf)
        pltpu.sync_copy(rk_hbm.at[pl.ds(pixb // 2 * (DPAD // 64), RKV_ROWS)], rkbuf)

        def pair(q, _):
            for half in range(2):
                i = q * 2 + half
                t0 = wtbuf[i, pl.ds(64, 16)]
                t1 = wtbuf[i, pl.ds(80, 16)]
                t2 = wtbuf[i, pl.ds(96, 16)]
                t3 = wtbuf[i, pl.ds(112, 16)]
                wv = [wtbuf[i, pl.ds(16 * g, 16)] for g in range(4)]
                for dd in range(DPAD):
                    wgt = wv[dd // 16][dd % 16]
                    row = half * DPAD + dd
                    scaled[row, pl.ds(0, 16)] = wgt * t0
                    scaled[row, pl.ds(16, 16)] = wgt * t1
                    scaled[row, pl.ds(32, 16)] = wgt * t2
                    scaled[row, pl.ds(48, 16)] = wgt * t3
            pltpu.sync_copy(scaled.at[pl.ds(0, 128)],
                            bev.at[rkbuf.at[q]], add=True)
            return 0
        lax.fori_loop(0, PCHUNK // 2, pair, 0)
    lax.fori_loop(0, NCHUNK, chunk, 0)

    plsc.subcore_barrier()

    obase = s * (ROWS_PER_TILE * C_OUT // 128)

    def rd(k, _):
        pltpu.sync_copy(bev.at[pl.ds(s * ROWS_PER_TILE + k * DPAD, DPAD)],
                        scaled.at[pl.ds(0, DPAD)])

        def rt(rr, _):
            for h in range(2):
                for j in range(4):
                    outbuf[rr, pl.ds(h * 64 + 16 * j, 16)] = (
                        scaled[rr * 2 + h, pl.ds(16 * j, 16)])
            return 0
        lax.fori_loop(0, 32, rt, 0)
        pltpu.sync_copy(outbuf, out_hbm.at[c, pl.ds(obase + k * 32, 32)])
        return 0
    lax.fori_loop(0, ROWS_PER_TILE // DPAD, rd, 0)


@_functools.cache
def _get_scatter():
    return pl.kernel(
        _sc_body,
        out_type=jax.ShapeDtypeStruct((NCORE, NV * C_OUT // 128, 128), jnp.float32),
        mesh=plsc.VectorSubcoreMesh(core_axis_name="c", subcore_axis_name="s",
                                    num_cores=NCORE, num_subcores=NTILE),
        compiler_params=pltpu.CompilerParams(use_tc_tiling_on_sc=False),
        scratch_types=[
            pltpu.VMEM_SHARED((NV, C_OUT), jnp.float32),
            pltpu.VMEM((PCHUNK, 128), jnp.float32),
            pltpu.VMEM((RKV_ROWS, 128), jnp.int32),
            pltpu.VMEM((128, C_OUT), jnp.float32),
            pltpu.VMEM((32, 128), jnp.float32),
        ],
    )


def _comb_body(parts_ref, eye_ref, out_ref):
    p = parts_ref[...]
    ssum = p[0] + p[1]
    out_ref[...] = lax.dot_general(eye_ref[...], ssum, (((1,), (1,)), ((), ())),
                                   preferred_element_type=jnp.float32)


_comb = pl.pallas_call(
    _comb_body,
    grid=(8,),
    in_specs=[
        pl.BlockSpec((NCORE, 2048, C_OUT), lambda k: (0, k, 0)),
        pl.BlockSpec((C_OUT, C_OUT), lambda k: (0, 0)),
    ],
    out_specs=pl.BlockSpec((C_OUT, 2048), lambda k: (0, k)),
    out_shape=jax.ShapeDtypeStruct((C_OUT, NVOX), jnp.float32),
)


def kernel(img_feats, rots, trans, intrins, post_rots, post_trans, W, b):
    B, N = img_feats.shape[:2]
    img = img_feats.reshape(B * N, 512, NPIX)
    Wp = jnp.zeros((128, 512), jnp.float32).at[:D + C_OUT].set(W)
    bp = jnp.zeros((1, 128), jnp.float32).at[0, :D + C_OUT].set(b)
    geom = _geometry(rots, trans, intrins, post_rots, post_trans)
    wt, rk = _prep(img, Wp, bp, geom)
    rkv = rk.reshape(NPIX_ALL * DPAD // 128, 128)
    parts = _get_scatter()(wt, rkv)
    parts = parts.reshape(NCORE, NV, C_OUT)
    eye = jnp.eye(C_OUT, dtype=jnp.float32)
    flat = _comb(parts, eye)
    return flat.reshape(1, C_OUT, 128, 128)
